# baseline (speedup 1.0000x reference)
import functools

import jax
import jax.numpy as jnp
from jax import lax
from jax.experimental import pallas as pl
from jax.experimental.pallas import tpu as pltpu
from jax.experimental.pallas import tpu_sc as plsc

N = 10000
E = 160000
A = 32
BD = 16
STEPS = 4
NP = 10016
CH = 128
NCHUNK = 1280
EPAD = NCHUNK * CH
CPW = NCHUNK // 32
EB = 512
NB = 2000
NBUF = 4

@functools.lru_cache(maxsize=1)
def _build_sc_kernels():
    mesh = plsc.VectorSubcoreMesh(core_axis_name="c", subcore_axis_name="s")
    params = pltpu.CompilerParams(use_tc_tiling_on_sc=False)

    @functools.partial(
        pl.kernel,
        out_type=jax.ShapeDtypeStruct((EPAD, A), jnp.float32),
        mesh=mesh,
        scratch_types=[
            pltpu.VMEM((CPW, CH), jnp.int32),
            pltpu.VMEM((NBUF, CH, A), jnp.float32),
            pltpu.SemaphoreType.DMA((NBUF,)),
            pltpu.SemaphoreType.DMA((NBUF,)),
        ],
        compiler_params=params,
    )
    def sc_gather(h_hbm, dst2_hbm, nbr_hbm, idx_buf, rows, gsem, ssem):
        c = lax.axis_index("c")
        s = lax.axis_index("s")
        wid = c * 16 + s
        pltpu.sync_copy(dst2_hbm.at[pl.ds(wid * CPW, CPW)], idx_buf)

        def group(g, carry):
            t0 = g * NBUF
            fetch = []
            for b in range(NBUF):
                fetch.append(pltpu.async_copy(
                    h_hbm.at[idx_buf.at[t0 + b]], rows.at[b], gsem.at[b]))
            store = []
            for b in range(NBUF):
                fetch[b].wait()
                base = (wid * CPW + t0 + b) * CH
                store.append(pltpu.async_copy(
                    rows.at[b], nbr_hbm.at[pl.ds(base, CH)], ssem.at[b]))
            for b in range(NBUF):
                store[b].wait()
            return carry

        lax.fori_loop(0, CPW // NBUF, group, 0)

    @functools.partial(
        pl.kernel,
        out_type=jax.ShapeDtypeStruct((2, NP, A), jnp.float32),
        mesh=mesh,
        scratch_types=[
            pltpu.VMEM((CPW, CH), jnp.int32),
            pltpu.VMEM((NBUF, CH, A), jnp.float32),
            pltpu.VMEM_SHARED((NP, A), jnp.float32),
            pltpu.SemaphoreType.DMA((NBUF,)),
            pltpu.SemaphoreType.DMA((NBUF,)),
        ],
        compiler_params=params,
    )
    def sc_scatter(msg_hbm, src2_hbm, zeros_hbm, parts_hbm, idx_buf, rows,
                   shared, gsem, ssem):
        c = lax.axis_index("c")
        s = lax.axis_index("s")
        wid = c * 16 + s

        @pl.when(s == 0)
        def _init():
            pltpu.sync_copy(zeros_hbm, shared)

        pltpu.sync_copy(src2_hbm.at[pl.ds(wid * CPW, CPW)], idx_buf)
        plsc.subcore_barrier()

        def group(g, carry):
            t0 = g * NBUF
            fetch = []
            for b in range(NBUF):
                base = (wid * CPW + t0 + b) * CH
                fetch.append(pltpu.async_copy(
                    msg_hbm.at[pl.ds(base, CH)], rows.at[b], gsem.at[b]))
            scat = []
            for b in range(NBUF):
                fetch[b].wait()
                scat.append(pltpu.async_copy(
                    rows.at[b], shared.at[idx_buf.at[t0 + b]], ssem.at[b],
                    add=True))
            for b in range(NBUF):
                scat[b].wait()
            return carry

        lax.fori_loop(0, CPW // NBUF, group, 0)
        plsc.subcore_barrier()

        @pl.when(s == 0)
        def _flush():
            pltpu.sync_copy(shared, parts_hbm.at[c])

    return sc_gather, sc_scatter


def _msg_body(nbr_ref, bond_ref, kall_ref, r1_ref, s_ref, msg_ref):
    nbr = nbr_ref[...].astype(jnp.bfloat16)
    bond1 = bond_ref[...]
    u = jnp.dot(nbr, kall_ref[...].astype(jnp.bfloat16),
                preferred_element_type=jnp.float32)
    brep = jnp.dot(bond1, r1_ref[...], preferred_element_type=jnp.float32)
    w = (u * brep).astype(jnp.bfloat16)
    msg_ref[...] = jnp.dot(w, s_ref[...].astype(jnp.bfloat16),
                           preferred_element_type=jnp.float32)


_msg_call = pl.pallas_call(
    _msg_body,
    grid=(EPAD // EB,),
    in_specs=[
        pl.BlockSpec((EB, A), lambda i: (i, 0)),
        pl.BlockSpec((EB, BD + 1), lambda i: (i, 0)),
        pl.BlockSpec((A, (BD + 1) * A), lambda i: (0, 0)),
        pl.BlockSpec((BD + 1, (BD + 1) * A), lambda i: (0, 0)),
        pl.BlockSpec(((BD + 1) * A, A), lambda i: (0, 0)),
    ],
    out_specs=pl.BlockSpec((EB, A), lambda i: (i, 0)),
    out_shape=jax.ShapeDtypeStruct((EPAD, A), jnp.float32),
)


def _gru_body(parts_ref, h_ref, wih_ref, whh_ref, bih_ref, bhh_ref, out_ref):
    x = parts_ref[0] + parts_ref[1]
    h = h_ref[...]
    gi = jnp.dot(x, wih_ref[...], preferred_element_type=jnp.float32,
                 precision=lax.Precision.HIGHEST) + bih_ref[...]
    gh = jnp.dot(h, whh_ref[...], preferred_element_type=jnp.float32,
                 precision=lax.Precision.HIGHEST) + bhh_ref[...]
    r = jax.nn.sigmoid(gi[:, 0:A] + gh[:, 0:A])
    z = jax.nn.sigmoid(gi[:, A:2 * A] + gh[:, A:2 * A])
    n = jnp.tanh(gi[:, 2 * A:3 * A] + r * gh[:, 2 * A:3 * A])
    out_ref[...] = (1.0 - z) * n + z * h


_gru_call = pl.pallas_call(
    _gru_body,
    grid=(N // NB,),
    in_specs=[
        pl.BlockSpec((2, NB, A), lambda i: (0, i, 0)),
        pl.BlockSpec((NB, A), lambda i: (i, 0)),
        pl.BlockSpec((A, 3 * A), lambda i: (0, 0)),
        pl.BlockSpec((A, 3 * A), lambda i: (0, 0)),
        pl.BlockSpec((1, 3 * A), lambda i: (0, 0)),
        pl.BlockSpec((1, 3 * A), lambda i: (0, 0)),
    ],
    out_specs=pl.BlockSpec((NB, A), lambda i: (i, 0)),
    out_shape=jax.ShapeDtypeStruct((N, A), jnp.float32),
)


def kernel(atom_features, bond_features, pair_indices, kernel, bias, w_ih, w_hh, b_ih, b_hh):
    src = pair_indices[:, 0]
    dst = pair_indices[:, 1]
    npad = EPAD - E
    src2 = jnp.concatenate(
        [src, jnp.full((npad,), N, jnp.int32)]).reshape(NCHUNK, CH)
    dst2 = jnp.concatenate(
        [dst, jnp.zeros((npad,), jnp.int32)]).reshape(NCHUNK, CH)
    bond1_pad = jnp.concatenate([
        jnp.ones((EPAD, 1), jnp.float32),
        jnp.concatenate(
            [bond_features, jnp.zeros((npad, BD), jnp.float32)]),
    ], axis=1)

    kr = kernel.reshape(BD, A, A)
    kt = kr.transpose(2, 0, 1).reshape(A, BD * A)
    bt = bias.reshape(A, A).T
    kall = jnp.concatenate([bt, kt], axis=1)
    eye = jnp.eye(A, dtype=jnp.float32)
    r1 = jnp.kron(jnp.eye(BD + 1, dtype=jnp.float32),
                  jnp.ones((1, A), jnp.float32))
    s_fold = jnp.tile(eye, (BD + 1, 1))

    wih_t = w_ih.T
    whh_t = w_hh.T
    bih = b_ih.reshape(1, 3 * A)
    bhh = b_hh.reshape(1, 3 * A)
    zeros_np = jnp.zeros((NP, A), jnp.float32)

    sc_gather, sc_scatter = _build_sc_kernels()
    h = atom_features
    for _ in range(STEPS):
        nbr = sc_gather(h, dst2)
        msg = _msg_call(nbr, bond1_pad, kall, r1, s_fold)
        parts = sc_scatter(msg, src2, zeros_np)
        h = _gru_call(parts, h, wih_t, whh_t, bih, bhh)
    return h


# speedup vs baseline: 1.0127x; 1.0127x over previous
import functools

import jax
import jax.numpy as jnp
from jax import lax
from jax.experimental import pallas as pl
from jax.experimental.pallas import tpu as pltpu
from jax.experimental.pallas import tpu_sc as plsc

N = 10000
E = 160000
A = 32
BD = 16
STEPS = 4
NP = 10016
CH = 128
NCHUNK = 1280
EPAD = NCHUNK * CH
CPW = NCHUNK // 32
EB = 512
NB = 2000
NBUF = 8

@functools.lru_cache(maxsize=1)
def _build_sc_kernels():
    mesh = plsc.VectorSubcoreMesh(core_axis_name="c", subcore_axis_name="s")
    params = pltpu.CompilerParams(use_tc_tiling_on_sc=False)

    @functools.partial(
        pl.kernel,
        out_type=jax.ShapeDtypeStruct((EPAD, A), jnp.float32),
        mesh=mesh,
        scratch_types=[
            pltpu.VMEM((CPW, CH), jnp.int32),
            pltpu.VMEM((NBUF, CH, A), jnp.float32),
            pltpu.SemaphoreType.DMA((NBUF,)),
            pltpu.SemaphoreType.DMA((NBUF,)),
        ],
        compiler_params=params,
    )
    def sc_gather(h_hbm, dst2_hbm, nbr_hbm, idx_buf, rows, gsem, ssem):
        c = lax.axis_index("c")
        s = lax.axis_index("s")
        wid = c * 16 + s
        pltpu.sync_copy(dst2_hbm.at[pl.ds(wid * CPW, CPW)], idx_buf)

        def group(g, carry):
            t0 = g * NBUF
            fetch = []
            for b in range(NBUF):
                fetch.append(pltpu.async_copy(
                    h_hbm.at[idx_buf.at[t0 + b]], rows.at[b], gsem.at[b]))
            store = []
            for b in range(NBUF):
                fetch[b].wait()
                base = (wid * CPW + t0 + b) * CH
                store.append(pltpu.async_copy(
                    rows.at[b], nbr_hbm.at[pl.ds(base, CH)], ssem.at[b]))
            for b in range(NBUF):
                store[b].wait()
            return carry

        lax.fori_loop(0, CPW // NBUF, group, 0)

    @functools.partial(
        pl.kernel,
        out_type=jax.ShapeDtypeStruct((2, NP, A), jnp.float32),
        mesh=mesh,
        scratch_types=[
            pltpu.VMEM((CPW, CH), jnp.int32),
            pltpu.VMEM((NBUF, CH, A), jnp.float32),
            pltpu.VMEM_SHARED((NP, A), jnp.float32),
            pltpu.SemaphoreType.DMA((NBUF,)),
            pltpu.SemaphoreType.DMA((NBUF,)),
        ],
        compiler_params=params,
    )
    def sc_scatter(msg_hbm, src2_hbm, zeros_hbm, parts_hbm, idx_buf, rows,
                   shared, gsem, ssem):
        c = lax.axis_index("c")
        s = lax.axis_index("s")
        wid = c * 16 + s

        @pl.when(s == 0)
        def _init():
            pltpu.sync_copy(zeros_hbm, shared)

        pltpu.sync_copy(src2_hbm.at[pl.ds(wid * CPW, CPW)], idx_buf)
        plsc.subcore_barrier()

        def group(g, carry):
            t0 = g * NBUF
            fetch = []
            for b in range(NBUF):
                base = (wid * CPW + t0 + b) * CH
                fetch.append(pltpu.async_copy(
                    msg_hbm.at[pl.ds(base, CH)], rows.at[b], gsem.at[b]))
            scat = []
            for b in range(NBUF):
                fetch[b].wait()
                scat.append(pltpu.async_copy(
                    rows.at[b], shared.at[idx_buf.at[t0 + b]], ssem.at[b],
                    add=True))
            for b in range(NBUF):
                scat[b].wait()
            return carry

        lax.fori_loop(0, CPW // NBUF, group, 0)
        plsc.subcore_barrier()

        @pl.when(s == 0)
        def _flush():
            pltpu.sync_copy(shared, parts_hbm.at[c])

    return sc_gather, sc_scatter


def _msg_body(nbr_ref, bond_ref, kall_ref, r1_ref, s_ref, msg_ref):
    nbr = nbr_ref[...].astype(jnp.bfloat16)
    bond1 = bond_ref[...]
    u = jnp.dot(nbr, kall_ref[...].astype(jnp.bfloat16),
                preferred_element_type=jnp.float32)
    brep = jnp.dot(bond1, r1_ref[...], preferred_element_type=jnp.float32)
    w = (u * brep).astype(jnp.bfloat16)
    msg_ref[...] = jnp.dot(w, s_ref[...].astype(jnp.bfloat16),
                           preferred_element_type=jnp.float32)


_msg_call = pl.pallas_call(
    _msg_body,
    grid=(EPAD // EB,),
    in_specs=[
        pl.BlockSpec((EB, A), lambda i: (i, 0)),
        pl.BlockSpec((EB, BD + 1), lambda i: (i, 0)),
        pl.BlockSpec((A, (BD + 1) * A), lambda i: (0, 0)),
        pl.BlockSpec((BD + 1, (BD + 1) * A), lambda i: (0, 0)),
        pl.BlockSpec(((BD + 1) * A, A), lambda i: (0, 0)),
    ],
    out_specs=pl.BlockSpec((EB, A), lambda i: (i, 0)),
    out_shape=jax.ShapeDtypeStruct((EPAD, A), jnp.float32),
)


def _gru_body(parts_ref, h_ref, wih_ref, whh_ref, bih_ref, bhh_ref, out_ref):
    x = parts_ref[0] + parts_ref[1]
    h = h_ref[...]
    gi = jnp.dot(x, wih_ref[...], preferred_element_type=jnp.float32,
                 precision=lax.Precision.HIGHEST) + bih_ref[...]
    gh = jnp.dot(h, whh_ref[...], preferred_element_type=jnp.float32,
                 precision=lax.Precision.HIGHEST) + bhh_ref[...]
    r = jax.nn.sigmoid(gi[:, 0:A] + gh[:, 0:A])
    z = jax.nn.sigmoid(gi[:, A:2 * A] + gh[:, A:2 * A])
    n = jnp.tanh(gi[:, 2 * A:3 * A] + r * gh[:, 2 * A:3 * A])
    out_ref[...] = (1.0 - z) * n + z * h


_gru_call = pl.pallas_call(
    _gru_body,
    grid=(N // NB,),
    in_specs=[
        pl.BlockSpec((2, NB, A), lambda i: (0, i, 0)),
        pl.BlockSpec((NB, A), lambda i: (i, 0)),
        pl.BlockSpec((A, 3 * A), lambda i: (0, 0)),
        pl.BlockSpec((A, 3 * A), lambda i: (0, 0)),
        pl.BlockSpec((1, 3 * A), lambda i: (0, 0)),
        pl.BlockSpec((1, 3 * A), lambda i: (0, 0)),
    ],
    out_specs=pl.BlockSpec((NB, A), lambda i: (i, 0)),
    out_shape=jax.ShapeDtypeStruct((N, A), jnp.float32),
)


def kernel(atom_features, bond_features, pair_indices, kernel, bias, w_ih, w_hh, b_ih, b_hh):
    src = pair_indices[:, 0]
    dst = pair_indices[:, 1]
    npad = EPAD - E
    src2 = jnp.concatenate(
        [src, jnp.full((npad,), N, jnp.int32)]).reshape(NCHUNK, CH)
    dst2 = jnp.concatenate(
        [dst, jnp.zeros((npad,), jnp.int32)]).reshape(NCHUNK, CH)
    bond1_pad = jnp.concatenate([
        jnp.ones((EPAD, 1), jnp.float32),
        jnp.concatenate(
            [bond_features, jnp.zeros((npad, BD), jnp.float32)]),
    ], axis=1)

    kr = kernel.reshape(BD, A, A)
    kt = kr.transpose(2, 0, 1).reshape(A, BD * A)
    bt = bias.reshape(A, A).T
    kall = jnp.concatenate([bt, kt], axis=1)
    eye = jnp.eye(A, dtype=jnp.float32)
    r1 = jnp.kron(jnp.eye(BD + 1, dtype=jnp.float32),
                  jnp.ones((1, A), jnp.float32))
    s_fold = jnp.tile(eye, (BD + 1, 1))

    wih_t = w_ih.T
    whh_t = w_hh.T
    bih = b_ih.reshape(1, 3 * A)
    bhh = b_hh.reshape(1, 3 * A)
    zeros_np = jnp.zeros((NP, A), jnp.float32)

    sc_gather, sc_scatter = _build_sc_kernels()
    h = atom_features
    for _ in range(STEPS):
        nbr = sc_gather(h, dst2)
        msg = _msg_call(nbr, bond1_pad, kall, r1, s_fold)
        parts = sc_scatter(msg, src2, zeros_np)
        h = _gru_call(parts, h, wih_t, whh_t, bih, bhh)
    return h
